# Initial kernel scaffold; baseline (speedup 1.0000x reference)
import jax
import jax.numpy as jnp
from jax.experimental import pallas as pl


def kernel(z, edge_index, W1l, b1, W1r, W2l, b2, W2r, W3l, b3, W3r, W4, b4):
    raise NotImplementedError("write your pallas kernel here")


# speedup vs baseline: 2.3794x; 2.3794x over previous
import functools

import jax
import jax.numpy as jnp
from jax import lax
from jax.experimental import pallas as pl
from jax.experimental.pallas import tpu as pltpu
from jax.experimental.pallas import tpu_sc as plsc

N_NODES = 50000
N_EDGES = 800000

NC = 2
NS = 16
NW = NC * NS

NP = 50176
ZROWS = NP // NS
EP = 819200
EPT = EP // NW
CHUNK = 128
NCHUNK = EPT // CHUNK
TRASH = NP - 1

ROW_TILE = 512
GRID = NP // ROW_TILE


def _make_agg(cb):
  mesh = plsc.VectorSubcoreMesh(core_axis_name="c", subcore_axis_name="s",
                                num_cores=NC, num_subcores=NS)

  @functools.partial(
      pl.kernel,
      out_type=jax.ShapeDtypeStruct((NC, NP, cb), jnp.float32),
      mesh=mesh,
      scratch_types=[
          pltpu.VMEM((CHUNK,), jnp.int32),
          pltpu.VMEM((CHUNK,), jnp.int32),
          pltpu.VMEM((CHUNK, cb), jnp.float32),
          pltpu.VMEM_SHARED((NP, cb), jnp.float32),
          pltpu.SemaphoreType.DMA,
      ],
      compiler_params=pltpu.CompilerParams(use_tc_tiling_on_sc=False),
  )
  def agg(x_hbm, src_hbm, dst_hbm, zeros_hbm, out_hbm,
          src_v, dst_v, rows_v, acc_sh, sem):
    c = lax.axis_index("c")
    s = lax.axis_index("s")
    wid = c * NS + s

    pltpu.sync_copy(zeros_hbm, acc_sh.at[pl.ds(s * ZROWS, ZROWS)])
    plsc.subcore_barrier()

    base = wid * EPT

    def body(i, carry):
      off = base + i * CHUNK
      pltpu.sync_copy(src_hbm.at[pl.ds(off, CHUNK)], src_v)
      pltpu.sync_copy(dst_hbm.at[pl.ds(off, CHUNK)], dst_v)
      pltpu.async_copy(x_hbm.at[src_v], rows_v, sem).wait()
      pltpu.sync_copy(rows_v, acc_sh.at[dst_v], add=True)
      return carry

    lax.fori_loop(0, NCHUNK, body, 0)
    plsc.subcore_barrier()

    pltpu.sync_copy(acc_sh.at[pl.ds(s * ZROWS, ZROWS)],
                    out_hbm.at[c, pl.ds(s * ZROWS, ZROWS)])

  return agg


_agg_cache = {}


def _agg16(*args):
  if 16 not in _agg_cache:
    _agg_cache[16] = _make_agg(16)
  return _agg_cache[16](*args)


def _agg32(*args):
  if 32 not in _agg_cache:
    _agg_cache[32] = _make_agg(32)
  return _agg_cache[32](*args)


def _d1_body(p1_ref, z_ref, wl_ref, wr_ref, b_ref, h1_ref, inv_ref):
  s = p1_ref[0] + p1_ref[1]
  cnt = s[:, 3:4]
  inv = 1.0 / jnp.maximum(cnt, 1.0)
  mean = s * inv
  h = (jnp.dot(mean, wl_ref[...], preferred_element_type=jnp.float32)
       + jnp.dot(z_ref[...], wr_ref[...], preferred_element_type=jnp.float32)
       + b_ref[...])
  h = jnp.maximum(h, 0.0)
  for p in range(4):
    h1_ref[p] = h[:, 32 * p:32 * (p + 1)]
  inv_ref[...] = jnp.broadcast_to(inv, (ROW_TILE, 8))


def _d1(p1, zp, wl, wr, b):
  return pl.pallas_call(
      _d1_body,
      grid=(GRID,),
      in_specs=[
          pl.BlockSpec((NC, ROW_TILE, 16), lambda i: (0, i, 0)),
          pl.BlockSpec((ROW_TILE, 16), lambda i: (i, 0)),
          pl.BlockSpec((16, 128), lambda i: (0, 0)),
          pl.BlockSpec((16, 128), lambda i: (0, 0)),
          pl.BlockSpec((1, 128), lambda i: (0, 0)),
      ],
      out_specs=[
          pl.BlockSpec((4, ROW_TILE, 32), lambda i: (0, i, 0)),
          pl.BlockSpec((ROW_TILE, 8), lambda i: (i, 0)),
      ],
      out_shape=[
          jax.ShapeDtypeStruct((4, NP, 32), jnp.float32),
          jax.ShapeDtypeStruct((NP, 8), jnp.float32),
      ],
  )(p1, zp, wl, wr, b)


def _d2_body(o0_ref, o1_ref, o2_ref, o3_ref, h1_ref, inv_ref,
             wl_ref, wr_ref, b_ref, w3l_ref, h2_ref, g3_ref):
  s2 = jnp.concatenate(
      [r[0] + r[1] for r in (o0_ref, o1_ref, o2_ref, o3_ref)], axis=1)
  mean2 = s2 * inv_ref[:, 0:1]
  h1 = jnp.concatenate([h1_ref[p] for p in range(4)], axis=1)
  h2 = (jnp.dot(mean2, wl_ref[...], preferred_element_type=jnp.float32)
        + jnp.dot(h1, wr_ref[...], preferred_element_type=jnp.float32)
        + b_ref[...])
  h2 = jnp.maximum(h2, 0.0)
  h2_ref[...] = h2
  g = jnp.dot(h2, w3l_ref[...], preferred_element_type=jnp.float32)
  g3_ref[0] = g[:, :32]
  g3_ref[1] = g[:, 32:]


def _d2(o2s, h1blk, inv8, wl, wr, b, w3l):
  spec_o = pl.BlockSpec((NC, ROW_TILE, 32), lambda i: (0, i, 0))
  return pl.pallas_call(
      _d2_body,
      grid=(GRID,),
      in_specs=[
          spec_o, spec_o, spec_o, spec_o,
          pl.BlockSpec((4, ROW_TILE, 32), lambda i: (0, i, 0)),
          pl.BlockSpec((ROW_TILE, 8), lambda i: (i, 0)),
          pl.BlockSpec((128, 128), lambda i: (0, 0)),
          pl.BlockSpec((128, 128), lambda i: (0, 0)),
          pl.BlockSpec((1, 128), lambda i: (0, 0)),
          pl.BlockSpec((128, 64), lambda i: (0, 0)),
      ],
      out_specs=[
          pl.BlockSpec((ROW_TILE, 128), lambda i: (i, 0)),
          pl.BlockSpec((2, ROW_TILE, 32), lambda i: (0, i, 0)),
      ],
      out_shape=[
          jax.ShapeDtypeStruct((NP, 128), jnp.float32),
          jax.ShapeDtypeStruct((2, NP, 32), jnp.float32),
      ],
  )(*o2s, h1blk, inv8, wl, wr, b, w3l)


def _d3_body(o0_ref, o1_ref, h2_ref, inv_ref, w3r_ref, b3_ref,
             w4_ref, b4_ref, out_ref):
  sg = jnp.concatenate([o0_ref[0] + o0_ref[1], o1_ref[0] + o1_ref[1]], axis=1)
  h3 = (sg * inv_ref[:, 0:1]
        + jnp.dot(h2_ref[...], w3r_ref[...], preferred_element_type=jnp.float32)
        + b3_ref[...])
  h3 = jnp.maximum(h3, 0.0)
  out_ref[...] = (jnp.dot(h3, w4_ref[...], preferred_element_type=jnp.float32)
                  + b4_ref[...])


def _d3(o3s, h2, inv8, w3r, b3, w4, b4):
  spec_o = pl.BlockSpec((NC, ROW_TILE, 32), lambda i: (0, i, 0))
  return pl.pallas_call(
      _d3_body,
      grid=(GRID,),
      in_specs=[
          spec_o, spec_o,
          pl.BlockSpec((ROW_TILE, 128), lambda i: (i, 0)),
          pl.BlockSpec((ROW_TILE, 8), lambda i: (i, 0)),
          pl.BlockSpec((128, 64), lambda i: (0, 0)),
          pl.BlockSpec((1, 64), lambda i: (0, 0)),
          pl.BlockSpec((64, 40), lambda i: (0, 0)),
          pl.BlockSpec((1, 40), lambda i: (0, 0)),
      ],
      out_specs=pl.BlockSpec((ROW_TILE, 40), lambda i: (i, 0)),
      out_shape=jax.ShapeDtypeStruct((NP, 40), jnp.float32),
  )(*o3s, h2, inv8, w3r, b3, w4, b4)


@jax.jit
def kernel(z, edge_index, W1l, b1, W1r, W2l, b2, W2r, W3l, b3, W3r, W4, b4):
  ei = edge_index.astype(jnp.int32)
  pad = EP - N_EDGES
  src = jnp.concatenate([ei[0], jnp.zeros((pad,), jnp.int32)])
  dst = jnp.concatenate([ei[1], jnp.full((pad,), TRASH, jnp.int32)])

  zp = jnp.zeros((NP, 16), jnp.float32)
  zp = zp.at[:N_NODES, :3].set(z)
  zp = zp.at[:N_NODES, 3].set(1.0)

  w1l = jnp.zeros((16, 128), jnp.float32).at[:3].set(W1l.T)
  w1r = jnp.zeros((16, 128), jnp.float32).at[:3].set(W1r.T)

  zer16 = jnp.zeros((ZROWS, 16), jnp.float32)
  zer32 = jnp.zeros((ZROWS, 32), jnp.float32)

  p1 = _agg16(zp, src, dst, zer16)
  h1blk, inv8 = _d1(p1, zp, w1l, w1r, b1.reshape(1, 128))

  o2s = [_agg32(h1blk[p], src, dst, zer32) for p in range(4)]
  h2, g3blk = _d2(o2s, h1blk, inv8, W2l.T, W2r.T, b2.reshape(1, 128), W3l.T)

  o3s = [_agg32(g3blk[p], src, dst, zer32) for p in range(2)]
  out = _d3(o3s, h2, inv8, W3r.T, b3.reshape(1, 64), W4.T, b4.reshape(1, 40))
  return out[:N_NODES]


# speedup vs baseline: 3.6716x; 1.5431x over previous
import functools

import jax
import jax.numpy as jnp
from jax import lax
from jax.experimental import pallas as pl
from jax.experimental.pallas import tpu as pltpu
from jax.experimental.pallas import tpu_sc as plsc

N_NODES = 50000
N_EDGES = 800000

NC = 2
NS = 16
NW = NC * NS

NP = 50176
ZROWS = NP // NS
EP = 819200
EPT = EP // NW
CHUNK = 128
NCHUNK = EPT // CHUNK
TRASH = NP - 1

ROW_TILE = 512
GRID = NP // ROW_TILE


NBUF = 4
NGROUP = NCHUNK // NBUF


def _make_agg(cb):
  mesh = plsc.VectorSubcoreMesh(core_axis_name="c", subcore_axis_name="s",
                                num_cores=NC, num_subcores=NS)

  @functools.partial(
      pl.kernel,
      out_type=jax.ShapeDtypeStruct((NC, NP, cb), jnp.float32),
      mesh=mesh,
      scratch_types=[
          pltpu.VMEM((2, NBUF, CHUNK), jnp.int32),
          pltpu.VMEM((2, NBUF, CHUNK), jnp.int32),
          pltpu.VMEM((NBUF, CHUNK, cb), jnp.float32),
          pltpu.VMEM_SHARED((NP, cb), jnp.float32),
      ] + [pltpu.SemaphoreType.DMA] * (2 * NBUF + 2),
      compiler_params=pltpu.CompilerParams(use_tc_tiling_on_sc=False),
  )
  def agg(x_hbm, src_hbm, dst_hbm, zeros_hbm, out_hbm,
          isrc, idst, rows, acc_sh, *sems):
    gsem = sems[:NBUF]
    ssem = sems[NBUF:2 * NBUF]
    xsem_s, xsem_d = sems[2 * NBUF], sems[2 * NBUF + 1]
    c = lax.axis_index("c")
    s = lax.axis_index("s")
    wid = c * NS + s

    pltpu.sync_copy(zeros_hbm, acc_sh.at[pl.ds(s * ZROWS, ZROWS)])
    pltpu.sync_copy(src_hbm.at[wid, pl.ds(0, NBUF)], isrc.at[0])
    pltpu.sync_copy(dst_hbm.at[wid, pl.ds(0, NBUF)], idst.at[0])
    plsc.subcore_barrier()

    def group(g, carry):
      p = lax.rem(g, 2)
      pn = 1 - p

      for b in range(NBUF):
        @pl.when(g > 0)
        def _():
          pltpu.make_async_copy(
              rows.at[b], acc_sh.at[idst.at[pn, b]], ssem[b]).wait()

      @pl.when(g + 1 < NGROUP)
      def _():
        nxt = (g + 1) * NBUF
        pltpu.async_copy(src_hbm.at[wid, pl.ds(nxt, NBUF)], isrc.at[pn],
                         xsem_s)
        pltpu.async_copy(dst_hbm.at[wid, pl.ds(nxt, NBUF)], idst.at[pn],
                         xsem_d)

      for b in range(NBUF):
        pltpu.async_copy(x_hbm.at[isrc.at[p, b]], rows.at[b], gsem[b])

      for b in range(NBUF):
        pltpu.make_async_copy(
            x_hbm.at[isrc.at[p, b]], rows.at[b], gsem[b]).wait()
        pltpu.async_copy(rows.at[b], acc_sh.at[idst.at[p, b]], ssem[b],
                         add=True)

      @pl.when(g + 1 < NGROUP)
      def _():
        nxt = (g + 1) * NBUF
        pltpu.make_async_copy(src_hbm.at[wid, pl.ds(nxt, NBUF)], isrc.at[pn],
                              xsem_s).wait()
        pltpu.make_async_copy(dst_hbm.at[wid, pl.ds(nxt, NBUF)], idst.at[pn],
                              xsem_d).wait()
      return carry

    lax.fori_loop(0, NGROUP, group, 0)
    pl_ = lax.rem(NGROUP - 1, 2)
    for b in range(NBUF):
      pltpu.make_async_copy(
          rows.at[b], acc_sh.at[idst.at[pl_, b]], ssem[b]).wait()
    plsc.subcore_barrier()

    pltpu.sync_copy(acc_sh.at[pl.ds(s * ZROWS, ZROWS)],
                    out_hbm.at[c, pl.ds(s * ZROWS, ZROWS)])

  return agg


_agg_cache = {}


def _agg16(*args):
  if 16 not in _agg_cache:
    _agg_cache[16] = _make_agg(16)
  return _agg_cache[16](*args)


def _agg32(*args):
  if 32 not in _agg_cache:
    _agg_cache[32] = _make_agg(32)
  return _agg_cache[32](*args)


def _d1_body(p1_ref, z_ref, wl_ref, wr_ref, b_ref, h1_ref, inv_ref):
  s = p1_ref[0] + p1_ref[1]
  cnt = s[:, 3:4]
  inv = 1.0 / jnp.maximum(cnt, 1.0)
  mean = s * inv
  h = (jnp.dot(mean, wl_ref[...], preferred_element_type=jnp.float32)
       + jnp.dot(z_ref[...], wr_ref[...], preferred_element_type=jnp.float32)
       + b_ref[...])
  h = jnp.maximum(h, 0.0)
  for p in range(4):
    h1_ref[p] = h[:, 32 * p:32 * (p + 1)]
  inv_ref[...] = jnp.broadcast_to(inv, (ROW_TILE, 8))


def _d1(p1, zp, wl, wr, b):
  return pl.pallas_call(
      _d1_body,
      grid=(GRID,),
      in_specs=[
          pl.BlockSpec((NC, ROW_TILE, 16), lambda i: (0, i, 0)),
          pl.BlockSpec((ROW_TILE, 16), lambda i: (i, 0)),
          pl.BlockSpec((16, 128), lambda i: (0, 0)),
          pl.BlockSpec((16, 128), lambda i: (0, 0)),
          pl.BlockSpec((1, 128), lambda i: (0, 0)),
      ],
      out_specs=[
          pl.BlockSpec((4, ROW_TILE, 32), lambda i: (0, i, 0)),
          pl.BlockSpec((ROW_TILE, 8), lambda i: (i, 0)),
      ],
      out_shape=[
          jax.ShapeDtypeStruct((4, NP, 32), jnp.float32),
          jax.ShapeDtypeStruct((NP, 8), jnp.float32),
      ],
  )(p1, zp, wl, wr, b)


def _d2_body(o0_ref, o1_ref, o2_ref, o3_ref, h1_ref, inv_ref,
             wl_ref, wr_ref, b_ref, w3l_ref, h2_ref, g3_ref):
  s2 = jnp.concatenate(
      [r[0] + r[1] for r in (o0_ref, o1_ref, o2_ref, o3_ref)], axis=1)
  mean2 = s2 * inv_ref[:, 0:1]
  h1 = jnp.concatenate([h1_ref[p] for p in range(4)], axis=1)
  h2 = (jnp.dot(mean2, wl_ref[...], preferred_element_type=jnp.float32)
        + jnp.dot(h1, wr_ref[...], preferred_element_type=jnp.float32)
        + b_ref[...])
  h2 = jnp.maximum(h2, 0.0)
  h2_ref[...] = h2
  g = jnp.dot(h2, w3l_ref[...], preferred_element_type=jnp.float32)
  g3_ref[0] = g[:, :32]
  g3_ref[1] = g[:, 32:]


def _d2(o2s, h1blk, inv8, wl, wr, b, w3l):
  spec_o = pl.BlockSpec((NC, ROW_TILE, 32), lambda i: (0, i, 0))
  return pl.pallas_call(
      _d2_body,
      grid=(GRID,),
      in_specs=[
          spec_o, spec_o, spec_o, spec_o,
          pl.BlockSpec((4, ROW_TILE, 32), lambda i: (0, i, 0)),
          pl.BlockSpec((ROW_TILE, 8), lambda i: (i, 0)),
          pl.BlockSpec((128, 128), lambda i: (0, 0)),
          pl.BlockSpec((128, 128), lambda i: (0, 0)),
          pl.BlockSpec((1, 128), lambda i: (0, 0)),
          pl.BlockSpec((128, 64), lambda i: (0, 0)),
      ],
      out_specs=[
          pl.BlockSpec((ROW_TILE, 128), lambda i: (i, 0)),
          pl.BlockSpec((2, ROW_TILE, 32), lambda i: (0, i, 0)),
      ],
      out_shape=[
          jax.ShapeDtypeStruct((NP, 128), jnp.float32),
          jax.ShapeDtypeStruct((2, NP, 32), jnp.float32),
      ],
  )(*o2s, h1blk, inv8, wl, wr, b, w3l)


def _d3_body(o0_ref, o1_ref, h2_ref, inv_ref, w3r_ref, b3_ref,
             w4_ref, b4_ref, out_ref):
  sg = jnp.concatenate([o0_ref[0] + o0_ref[1], o1_ref[0] + o1_ref[1]], axis=1)
  h3 = (sg * inv_ref[:, 0:1]
        + jnp.dot(h2_ref[...], w3r_ref[...], preferred_element_type=jnp.float32)
        + b3_ref[...])
  h3 = jnp.maximum(h3, 0.0)
  out_ref[...] = (jnp.dot(h3, w4_ref[...], preferred_element_type=jnp.float32)
                  + b4_ref[...])


def _d3(o3s, h2, inv8, w3r, b3, w4, b4):
  spec_o = pl.BlockSpec((NC, ROW_TILE, 32), lambda i: (0, i, 0))
  return pl.pallas_call(
      _d3_body,
      grid=(GRID,),
      in_specs=[
          spec_o, spec_o,
          pl.BlockSpec((ROW_TILE, 128), lambda i: (i, 0)),
          pl.BlockSpec((ROW_TILE, 8), lambda i: (i, 0)),
          pl.BlockSpec((128, 64), lambda i: (0, 0)),
          pl.BlockSpec((1, 64), lambda i: (0, 0)),
          pl.BlockSpec((64, 40), lambda i: (0, 0)),
          pl.BlockSpec((1, 40), lambda i: (0, 0)),
      ],
      out_specs=pl.BlockSpec((ROW_TILE, 40), lambda i: (i, 0)),
      out_shape=jax.ShapeDtypeStruct((NP, 40), jnp.float32),
  )(*o3s, h2, inv8, w3r, b3, w4, b4)


@jax.jit
def kernel(z, edge_index, W1l, b1, W1r, W2l, b2, W2r, W3l, b3, W3r, W4, b4):
  ei = edge_index.astype(jnp.int32)
  pad = EP - N_EDGES
  src = jnp.concatenate([ei[0], jnp.zeros((pad,), jnp.int32)])
  dst = jnp.concatenate([ei[1], jnp.full((pad,), TRASH, jnp.int32)])
  src = src.reshape(NW, NCHUNK, CHUNK)
  dst = dst.reshape(NW, NCHUNK, CHUNK)

  zp = jnp.zeros((NP, 16), jnp.float32)
  zp = zp.at[:N_NODES, :3].set(z)
  zp = zp.at[:N_NODES, 3].set(1.0)

  w1l = jnp.zeros((16, 128), jnp.float32).at[:3].set(W1l.T)
  w1r = jnp.zeros((16, 128), jnp.float32).at[:3].set(W1r.T)

  zer16 = jnp.zeros((ZROWS, 16), jnp.float32)
  zer32 = jnp.zeros((ZROWS, 32), jnp.float32)

  p1 = _agg16(zp, src, dst, zer16)
  h1blk, inv8 = _d1(p1, zp, w1l, w1r, b1.reshape(1, 128))

  o2s = [_agg32(h1blk[p], src, dst, zer32) for p in range(4)]
  h2, g3blk = _d2(o2s, h1blk, inv8, W2l.T, W2r.T, b2.reshape(1, 128), W3l.T)

  o3s = [_agg32(g3blk[p], src, dst, zer32) for p in range(2)]
  out = _d3(o3s, h2, inv8, W3r.T, b3.reshape(1, 64), W4.T, b4.reshape(1, 40))
  return out[:N_NODES]


# speedup vs baseline: 3.7376x; 1.0180x over previous
import functools

import jax
import jax.numpy as jnp
from jax import lax
from jax.experimental import pallas as pl
from jax.experimental.pallas import tpu as pltpu
from jax.experimental.pallas import tpu_sc as plsc

N_NODES = 50000
N_EDGES = 800000

NC = 2
NS = 16
NW = NC * NS

NP = 50176
ZROWS = NP // NS
EP = 819200
EPT = EP // NW
CHUNK = 128
NCHUNK = EPT // CHUNK
TRASH = NP - 1

ROW_TILE = 512
GRID = NP // ROW_TILE


NBUF = 4
NGROUP = NCHUNK // NBUF


def _make_agg(cb):
  mesh = plsc.VectorSubcoreMesh(core_axis_name="c", subcore_axis_name="s",
                                num_cores=NC, num_subcores=NS)

  @functools.partial(
      pl.kernel,
      out_type=jax.ShapeDtypeStruct((NC, NP, cb), jnp.float32),
      mesh=mesh,
      scratch_types=[
          pltpu.VMEM((2, NBUF, CHUNK), jnp.int32),
          pltpu.VMEM((2, NBUF, CHUNK), jnp.int32),
          pltpu.VMEM((NBUF, CHUNK, cb), jnp.float32),
          pltpu.VMEM_SHARED((NP, cb), jnp.float32),
      ] + [pltpu.SemaphoreType.DMA] * (2 * NBUF + 2),
      compiler_params=pltpu.CompilerParams(use_tc_tiling_on_sc=False),
  )
  def agg(x_hbm, src_hbm, dst_hbm, zeros_hbm, out_hbm,
          isrc, idst, rows, acc_sh, *sems):
    gsem = sems[:NBUF]
    ssem = sems[NBUF:2 * NBUF]
    xsem_s, xsem_d = sems[2 * NBUF], sems[2 * NBUF + 1]
    c = lax.axis_index("c")
    s = lax.axis_index("s")
    wid = c * NS + s

    pltpu.sync_copy(zeros_hbm, acc_sh.at[pl.ds(s * ZROWS, ZROWS)])
    pltpu.sync_copy(src_hbm.at[wid, pl.ds(0, NBUF)], isrc.at[0])
    pltpu.sync_copy(dst_hbm.at[wid, pl.ds(0, NBUF)], idst.at[0])
    plsc.subcore_barrier()

    def group(g, carry):
      p = lax.rem(g, 2)
      pn = 1 - p

      for b in range(NBUF):
        @pl.when(g > 0)
        def _():
          pltpu.make_async_copy(
              rows.at[b], acc_sh.at[idst.at[pn, b]], ssem[b]).wait()
        pltpu.async_copy(x_hbm.at[isrc.at[p, b]], rows.at[b], gsem[b])

      @pl.when(g + 1 < NGROUP)
      def _():
        nxt = (g + 1) * NBUF
        pltpu.async_copy(src_hbm.at[wid, pl.ds(nxt, NBUF)], isrc.at[pn],
                         xsem_s)
        pltpu.async_copy(dst_hbm.at[wid, pl.ds(nxt, NBUF)], idst.at[pn],
                         xsem_d)

      for b in range(NBUF):
        pltpu.make_async_copy(
            x_hbm.at[isrc.at[p, b]], rows.at[b], gsem[b]).wait()
        pltpu.async_copy(rows.at[b], acc_sh.at[idst.at[p, b]], ssem[b],
                         add=True)

      @pl.when(g + 1 < NGROUP)
      def _():
        nxt = (g + 1) * NBUF
        pltpu.make_async_copy(src_hbm.at[wid, pl.ds(nxt, NBUF)], isrc.at[pn],
                              xsem_s).wait()
        pltpu.make_async_copy(dst_hbm.at[wid, pl.ds(nxt, NBUF)], idst.at[pn],
                              xsem_d).wait()
      return carry

    lax.fori_loop(0, NGROUP, group, 0)
    pl_ = lax.rem(NGROUP - 1, 2)
    for b in range(NBUF):
      pltpu.make_async_copy(
          rows.at[b], acc_sh.at[idst.at[pl_, b]], ssem[b]).wait()
    plsc.subcore_barrier()

    pltpu.sync_copy(acc_sh.at[pl.ds(s * ZROWS, ZROWS)],
                    out_hbm.at[c, pl.ds(s * ZROWS, ZROWS)])

  return agg


_agg_cache = {}


def _agg16(*args):
  if 16 not in _agg_cache:
    _agg_cache[16] = _make_agg(16)
  return _agg_cache[16](*args)


def _agg32(*args):
  if 32 not in _agg_cache:
    _agg_cache[32] = _make_agg(32)
  return _agg_cache[32](*args)


def _d1_body(p1_ref, z_ref, wl_ref, wr_ref, b_ref, h1_ref, inv_ref):
  s = p1_ref[0] + p1_ref[1]
  cnt = s[:, 3:4]
  inv = 1.0 / jnp.maximum(cnt, 1.0)
  mean = s * inv
  h = (jnp.dot(mean, wl_ref[...], preferred_element_type=jnp.float32)
       + jnp.dot(z_ref[...], wr_ref[...], preferred_element_type=jnp.float32)
       + b_ref[...])
  h = jnp.maximum(h, 0.0)
  for p in range(4):
    h1_ref[p] = h[:, 32 * p:32 * (p + 1)]
  inv_ref[...] = jnp.broadcast_to(inv, (ROW_TILE, 8))


def _d1(p1, zp, wl, wr, b):
  return pl.pallas_call(
      _d1_body,
      grid=(GRID,),
      in_specs=[
          pl.BlockSpec((NC, ROW_TILE, 16), lambda i: (0, i, 0)),
          pl.BlockSpec((ROW_TILE, 16), lambda i: (i, 0)),
          pl.BlockSpec((16, 128), lambda i: (0, 0)),
          pl.BlockSpec((16, 128), lambda i: (0, 0)),
          pl.BlockSpec((1, 128), lambda i: (0, 0)),
      ],
      out_specs=[
          pl.BlockSpec((4, ROW_TILE, 32), lambda i: (0, i, 0)),
          pl.BlockSpec((ROW_TILE, 8), lambda i: (i, 0)),
      ],
      out_shape=[
          jax.ShapeDtypeStruct((4, NP, 32), jnp.float32),
          jax.ShapeDtypeStruct((NP, 8), jnp.float32),
      ],
  )(p1, zp, wl, wr, b)


def _d2_body(o0_ref, o1_ref, o2_ref, o3_ref, h1_ref, inv_ref,
             wl_ref, wr_ref, b_ref, w3l_ref, h2_ref, g3_ref):
  s2 = jnp.concatenate(
      [r[0] + r[1] for r in (o0_ref, o1_ref, o2_ref, o3_ref)], axis=1)
  mean2 = s2 * inv_ref[:, 0:1]
  h1 = jnp.concatenate([h1_ref[p] for p in range(4)], axis=1)
  h2 = (jnp.dot(mean2, wl_ref[...], preferred_element_type=jnp.float32)
        + jnp.dot(h1, wr_ref[...], preferred_element_type=jnp.float32)
        + b_ref[...])
  h2 = jnp.maximum(h2, 0.0)
  h2_ref[...] = h2
  g = jnp.dot(h2, w3l_ref[...], preferred_element_type=jnp.float32)
  g3_ref[0] = g[:, :32]
  g3_ref[1] = g[:, 32:]


def _d2(o2s, h1blk, inv8, wl, wr, b, w3l):
  spec_o = pl.BlockSpec((NC, ROW_TILE, 32), lambda i: (0, i, 0))
  return pl.pallas_call(
      _d2_body,
      grid=(GRID,),
      in_specs=[
          spec_o, spec_o, spec_o, spec_o,
          pl.BlockSpec((4, ROW_TILE, 32), lambda i: (0, i, 0)),
          pl.BlockSpec((ROW_TILE, 8), lambda i: (i, 0)),
          pl.BlockSpec((128, 128), lambda i: (0, 0)),
          pl.BlockSpec((128, 128), lambda i: (0, 0)),
          pl.BlockSpec((1, 128), lambda i: (0, 0)),
          pl.BlockSpec((128, 64), lambda i: (0, 0)),
      ],
      out_specs=[
          pl.BlockSpec((ROW_TILE, 128), lambda i: (i, 0)),
          pl.BlockSpec((2, ROW_TILE, 32), lambda i: (0, i, 0)),
      ],
      out_shape=[
          jax.ShapeDtypeStruct((NP, 128), jnp.float32),
          jax.ShapeDtypeStruct((2, NP, 32), jnp.float32),
      ],
  )(*o2s, h1blk, inv8, wl, wr, b, w3l)


def _d3_body(o0_ref, o1_ref, h2_ref, inv_ref, w3r_ref, b3_ref,
             w4_ref, b4_ref, out_ref):
  sg = jnp.concatenate([o0_ref[0] + o0_ref[1], o1_ref[0] + o1_ref[1]], axis=1)
  h3 = (sg * inv_ref[:, 0:1]
        + jnp.dot(h2_ref[...], w3r_ref[...], preferred_element_type=jnp.float32)
        + b3_ref[...])
  h3 = jnp.maximum(h3, 0.0)
  out_ref[...] = (jnp.dot(h3, w4_ref[...], preferred_element_type=jnp.float32)
                  + b4_ref[...])


def _d3(o3s, h2, inv8, w3r, b3, w4, b4):
  spec_o = pl.BlockSpec((NC, ROW_TILE, 32), lambda i: (0, i, 0))
  return pl.pallas_call(
      _d3_body,
      grid=(GRID,),
      in_specs=[
          spec_o, spec_o,
          pl.BlockSpec((ROW_TILE, 128), lambda i: (i, 0)),
          pl.BlockSpec((ROW_TILE, 8), lambda i: (i, 0)),
          pl.BlockSpec((128, 64), lambda i: (0, 0)),
          pl.BlockSpec((1, 64), lambda i: (0, 0)),
          pl.BlockSpec((64, 40), lambda i: (0, 0)),
          pl.BlockSpec((1, 40), lambda i: (0, 0)),
      ],
      out_specs=pl.BlockSpec((ROW_TILE, 40), lambda i: (i, 0)),
      out_shape=jax.ShapeDtypeStruct((NP, 40), jnp.float32),
  )(*o3s, h2, inv8, w3r, b3, w4, b4)


@jax.jit
def kernel(z, edge_index, W1l, b1, W1r, W2l, b2, W2r, W3l, b3, W3r, W4, b4):
  ei = edge_index.astype(jnp.int32)
  pad = EP - N_EDGES
  src = jnp.concatenate([ei[0], jnp.zeros((pad,), jnp.int32)])
  dst = jnp.concatenate([ei[1], jnp.full((pad,), TRASH, jnp.int32)])
  src = src.reshape(NW, NCHUNK, CHUNK)
  dst = dst.reshape(NW, NCHUNK, CHUNK)

  zp = jnp.zeros((NP, 16), jnp.float32)
  zp = zp.at[:N_NODES, :3].set(z)
  zp = zp.at[:N_NODES, 3].set(1.0)

  w1l = jnp.zeros((16, 128), jnp.float32).at[:3].set(W1l.T)
  w1r = jnp.zeros((16, 128), jnp.float32).at[:3].set(W1r.T)

  zer16 = jnp.zeros((ZROWS, 16), jnp.float32)
  zer32 = jnp.zeros((ZROWS, 32), jnp.float32)

  p1 = _agg16(zp, src, dst, zer16)
  h1blk, inv8 = _d1(p1, zp, w1l, w1r, b1.reshape(1, 128))

  o2s = [_agg32(h1blk[p], src, dst, zer32) for p in range(4)]
  h2, g3blk = _d2(o2s, h1blk, inv8, W2l.T, W2r.T, b2.reshape(1, 128), W3l.T)

  o3s = [_agg32(g3blk[p], src, dst, zer32) for p in range(2)]
  out = _d3(o3s, h2, inv8, W3r.T, b3.reshape(1, 64), W4.T, b4.reshape(1, 40))
  return out[:N_NODES]


# speedup vs baseline: 3.8103x; 1.0194x over previous
import functools

import jax
import jax.numpy as jnp
from jax import lax
from jax.experimental import pallas as pl
from jax.experimental.pallas import tpu as pltpu
from jax.experimental.pallas import tpu_sc as plsc

N_NODES = 50000
N_EDGES = 800000

NC = 2
NS = 16
NW = NC * NS

NP = 50176
ZROWS = NP // NS
EP = 819200
EPT = EP // NW
CHUNK = 256
NCHUNK = EPT // CHUNK
TRASH = NP - 1

ROW_TILE = 512
GRID = NP // ROW_TILE


NBUF = 2
NGROUP = NCHUNK // NBUF


def _make_agg(cb):
  mesh = plsc.VectorSubcoreMesh(core_axis_name="c", subcore_axis_name="s",
                                num_cores=NC, num_subcores=NS)

  @functools.partial(
      pl.kernel,
      out_type=jax.ShapeDtypeStruct((NC, NP, cb), jnp.float32),
      mesh=mesh,
      scratch_types=[
          pltpu.VMEM((2, NBUF, CHUNK), jnp.int32),
          pltpu.VMEM((2, NBUF, CHUNK), jnp.int32),
          pltpu.VMEM((NBUF, CHUNK, cb), jnp.float32),
          pltpu.VMEM_SHARED((NP, cb), jnp.float32),
      ] + [pltpu.SemaphoreType.DMA] * (2 * NBUF + 2),
      compiler_params=pltpu.CompilerParams(use_tc_tiling_on_sc=False),
  )
  def agg(x_hbm, src_hbm, dst_hbm, zeros_hbm, out_hbm,
          isrc, idst, rows, acc_sh, *sems):
    gsem = sems[:NBUF]
    ssem = sems[NBUF:2 * NBUF]
    xsem_s, xsem_d = sems[2 * NBUF], sems[2 * NBUF + 1]
    c = lax.axis_index("c")
    s = lax.axis_index("s")
    wid = c * NS + s

    pltpu.sync_copy(zeros_hbm, acc_sh.at[pl.ds(s * ZROWS, ZROWS)])
    pltpu.sync_copy(src_hbm.at[wid, pl.ds(0, NBUF)], isrc.at[0])
    pltpu.sync_copy(dst_hbm.at[wid, pl.ds(0, NBUF)], idst.at[0])
    plsc.subcore_barrier()

    def group(g, carry):
      p = lax.rem(g, 2)
      pn = 1 - p

      for b in range(NBUF):
        @pl.when(g > 0)
        def _():
          pltpu.make_async_copy(
              rows.at[b], acc_sh.at[idst.at[pn, b]], ssem[b]).wait()
        pltpu.async_copy(x_hbm.at[isrc.at[p, b]], rows.at[b], gsem[b])

      @pl.when(g + 1 < NGROUP)
      def _():
        nxt = (g + 1) * NBUF
        pltpu.async_copy(src_hbm.at[wid, pl.ds(nxt, NBUF)], isrc.at[pn],
                         xsem_s)
        pltpu.async_copy(dst_hbm.at[wid, pl.ds(nxt, NBUF)], idst.at[pn],
                         xsem_d)

      for b in range(NBUF):
        pltpu.make_async_copy(
            x_hbm.at[isrc.at[p, b]], rows.at[b], gsem[b]).wait()
        pltpu.async_copy(rows.at[b], acc_sh.at[idst.at[p, b]], ssem[b],
                         add=True)

      @pl.when(g + 1 < NGROUP)
      def _():
        nxt = (g + 1) * NBUF
        pltpu.make_async_copy(src_hbm.at[wid, pl.ds(nxt, NBUF)], isrc.at[pn],
                              xsem_s).wait()
        pltpu.make_async_copy(dst_hbm.at[wid, pl.ds(nxt, NBUF)], idst.at[pn],
                              xsem_d).wait()
      return carry

    lax.fori_loop(0, NGROUP, group, 0)
    pl_ = lax.rem(NGROUP - 1, 2)
    for b in range(NBUF):
      pltpu.make_async_copy(
          rows.at[b], acc_sh.at[idst.at[pl_, b]], ssem[b]).wait()
    plsc.subcore_barrier()

    pltpu.sync_copy(acc_sh.at[pl.ds(s * ZROWS, ZROWS)],
                    out_hbm.at[c, pl.ds(s * ZROWS, ZROWS)])

  return agg


_agg_cache = {}


def _agg16(*args):
  if 16 not in _agg_cache:
    _agg_cache[16] = _make_agg(16)
  return _agg_cache[16](*args)


def _agg32(*args):
  if 32 not in _agg_cache:
    _agg_cache[32] = _make_agg(32)
  return _agg_cache[32](*args)


def _d1_body(p1_ref, z_ref, wl_ref, wr_ref, b_ref, h1_ref, inv_ref):
  s = p1_ref[0] + p1_ref[1]
  cnt = s[:, 3:4]
  inv = 1.0 / jnp.maximum(cnt, 1.0)
  mean = s * inv
  h = (jnp.dot(mean, wl_ref[...], preferred_element_type=jnp.float32)
       + jnp.dot(z_ref[...], wr_ref[...], preferred_element_type=jnp.float32)
       + b_ref[...])
  h = jnp.maximum(h, 0.0)
  for p in range(4):
    h1_ref[p] = h[:, 32 * p:32 * (p + 1)]
  inv_ref[...] = jnp.broadcast_to(inv, (ROW_TILE, 8))


def _d1(p1, zp, wl, wr, b):
  return pl.pallas_call(
      _d1_body,
      grid=(GRID,),
      in_specs=[
          pl.BlockSpec((NC, ROW_TILE, 16), lambda i: (0, i, 0)),
          pl.BlockSpec((ROW_TILE, 16), lambda i: (i, 0)),
          pl.BlockSpec((16, 128), lambda i: (0, 0)),
          pl.BlockSpec((16, 128), lambda i: (0, 0)),
          pl.BlockSpec((1, 128), lambda i: (0, 0)),
      ],
      out_specs=[
          pl.BlockSpec((4, ROW_TILE, 32), lambda i: (0, i, 0)),
          pl.BlockSpec((ROW_TILE, 8), lambda i: (i, 0)),
      ],
      out_shape=[
          jax.ShapeDtypeStruct((4, NP, 32), jnp.float32),
          jax.ShapeDtypeStruct((NP, 8), jnp.float32),
      ],
  )(p1, zp, wl, wr, b)


def _d2_body(o0_ref, o1_ref, o2_ref, o3_ref, h1_ref, inv_ref,
             wl_ref, wr_ref, b_ref, w3l_ref, h2_ref, g3_ref):
  s2 = jnp.concatenate(
      [r[0] + r[1] for r in (o0_ref, o1_ref, o2_ref, o3_ref)], axis=1)
  mean2 = s2 * inv_ref[:, 0:1]
  h1 = jnp.concatenate([h1_ref[p] for p in range(4)], axis=1)
  h2 = (jnp.dot(mean2, wl_ref[...], preferred_element_type=jnp.float32)
        + jnp.dot(h1, wr_ref[...], preferred_element_type=jnp.float32)
        + b_ref[...])
  h2 = jnp.maximum(h2, 0.0)
  h2_ref[...] = h2
  g = jnp.dot(h2, w3l_ref[...], preferred_element_type=jnp.float32)
  g3_ref[0] = g[:, :32]
  g3_ref[1] = g[:, 32:]


def _d2(o2s, h1blk, inv8, wl, wr, b, w3l):
  spec_o = pl.BlockSpec((NC, ROW_TILE, 32), lambda i: (0, i, 0))
  return pl.pallas_call(
      _d2_body,
      grid=(GRID,),
      in_specs=[
          spec_o, spec_o, spec_o, spec_o,
          pl.BlockSpec((4, ROW_TILE, 32), lambda i: (0, i, 0)),
          pl.BlockSpec((ROW_TILE, 8), lambda i: (i, 0)),
          pl.BlockSpec((128, 128), lambda i: (0, 0)),
          pl.BlockSpec((128, 128), lambda i: (0, 0)),
          pl.BlockSpec((1, 128), lambda i: (0, 0)),
          pl.BlockSpec((128, 64), lambda i: (0, 0)),
      ],
      out_specs=[
          pl.BlockSpec((ROW_TILE, 128), lambda i: (i, 0)),
          pl.BlockSpec((2, ROW_TILE, 32), lambda i: (0, i, 0)),
      ],
      out_shape=[
          jax.ShapeDtypeStruct((NP, 128), jnp.float32),
          jax.ShapeDtypeStruct((2, NP, 32), jnp.float32),
      ],
  )(*o2s, h1blk, inv8, wl, wr, b, w3l)


def _d3_body(o0_ref, o1_ref, h2_ref, inv_ref, w3r_ref, b3_ref,
             w4_ref, b4_ref, out_ref):
  sg = jnp.concatenate([o0_ref[0] + o0_ref[1], o1_ref[0] + o1_ref[1]], axis=1)
  h3 = (sg * inv_ref[:, 0:1]
        + jnp.dot(h2_ref[...], w3r_ref[...], preferred_element_type=jnp.float32)
        + b3_ref[...])
  h3 = jnp.maximum(h3, 0.0)
  out_ref[...] = (jnp.dot(h3, w4_ref[...], preferred_element_type=jnp.float32)
                  + b4_ref[...])


def _d3(o3s, h2, inv8, w3r, b3, w4, b4):
  spec_o = pl.BlockSpec((NC, ROW_TILE, 32), lambda i: (0, i, 0))
  return pl.pallas_call(
      _d3_body,
      grid=(GRID,),
      in_specs=[
          spec_o, spec_o,
          pl.BlockSpec((ROW_TILE, 128), lambda i: (i, 0)),
          pl.BlockSpec((ROW_TILE, 8), lambda i: (i, 0)),
          pl.BlockSpec((128, 64), lambda i: (0, 0)),
          pl.BlockSpec((1, 64), lambda i: (0, 0)),
          pl.BlockSpec((64, 40), lambda i: (0, 0)),
          pl.BlockSpec((1, 40), lambda i: (0, 0)),
      ],
      out_specs=pl.BlockSpec((ROW_TILE, 40), lambda i: (i, 0)),
      out_shape=jax.ShapeDtypeStruct((NP, 40), jnp.float32),
  )(*o3s, h2, inv8, w3r, b3, w4, b4)


@jax.jit
def kernel(z, edge_index, W1l, b1, W1r, W2l, b2, W2r, W3l, b3, W3r, W4, b4):
  ei = edge_index.astype(jnp.int32)
  pad = EP - N_EDGES
  src = jnp.concatenate([ei[0], jnp.zeros((pad,), jnp.int32)])
  dst = jnp.concatenate([ei[1], jnp.full((pad,), TRASH, jnp.int32)])
  src = src.reshape(NW, NCHUNK, CHUNK)
  dst = dst.reshape(NW, NCHUNK, CHUNK)

  zp = jnp.zeros((NP, 16), jnp.float32)
  zp = zp.at[:N_NODES, :3].set(z)
  zp = zp.at[:N_NODES, 3].set(1.0)

  w1l = jnp.zeros((16, 128), jnp.float32).at[:3].set(W1l.T)
  w1r = jnp.zeros((16, 128), jnp.float32).at[:3].set(W1r.T)

  zer16 = jnp.zeros((ZROWS, 16), jnp.float32)
  zer32 = jnp.zeros((ZROWS, 32), jnp.float32)

  p1 = _agg16(zp, src, dst, zer16)
  h1blk, inv8 = _d1(p1, zp, w1l, w1r, b1.reshape(1, 128))

  o2s = [_agg32(h1blk[p], src, dst, zer32) for p in range(4)]
  h2, g3blk = _d2(o2s, h1blk, inv8, W2l.T, W2r.T, b2.reshape(1, 128), W3l.T)

  o3s = [_agg32(g3blk[p], src, dst, zer32) for p in range(2)]
  out = _d3(o3s, h2, inv8, W3r.T, b3.reshape(1, 64), W4.T, b4.reshape(1, 40))
  return out[:N_NODES]


# speedup vs baseline: 5.8541x; 1.5364x over previous
import functools

import jax
import jax.numpy as jnp
from jax import lax
from jax.experimental import pallas as pl
from jax.experimental.pallas import tpu as pltpu
from jax.experimental.pallas import tpu_sc as plsc

N_NODES = 50000
N_EDGES = 800000

NC = 2
NS = 16
NW = NC * NS

NP = 50176
ZROWS = NP // NS
EP = 819200
EPT = EP // NW
CHUNK = 256
NCHUNK = EPT // CHUNK
TRASH = NP - 1

ROW_TILE = 512
GRID = NP // ROW_TILE


NBUF = 2
NGROUP = NCHUNK // NBUF


def _make_agg(cb, dtype=jnp.float32):
  mesh = plsc.VectorSubcoreMesh(core_axis_name="c", subcore_axis_name="s",
                                num_cores=NC, num_subcores=NS)

  @functools.partial(
      pl.kernel,
      out_type=jax.ShapeDtypeStruct((NC, NP, cb), dtype),
      mesh=mesh,
      scratch_types=[
          pltpu.VMEM((2, NBUF, CHUNK), jnp.int32),
          pltpu.VMEM((2, NBUF, CHUNK), jnp.int32),
          pltpu.VMEM((NBUF, CHUNK, cb), dtype),
          pltpu.VMEM_SHARED((NP, cb), dtype),
      ] + [pltpu.SemaphoreType.DMA] * (2 * NBUF + 2),
      compiler_params=pltpu.CompilerParams(use_tc_tiling_on_sc=False),
  )
  def agg(x_hbm, src_hbm, dst_hbm, zeros_hbm, out_hbm,
          isrc, idst, rows, acc_sh, *sems):
    gsem = sems[:NBUF]
    ssem = sems[NBUF:2 * NBUF]
    xsem_s, xsem_d = sems[2 * NBUF], sems[2 * NBUF + 1]
    c = lax.axis_index("c")
    s = lax.axis_index("s")
    wid = c * NS + s

    pltpu.sync_copy(zeros_hbm, acc_sh.at[pl.ds(s * ZROWS, ZROWS)])
    pltpu.sync_copy(src_hbm.at[wid, pl.ds(0, NBUF)], isrc.at[0])
    pltpu.sync_copy(dst_hbm.at[wid, pl.ds(0, NBUF)], idst.at[0])
    plsc.subcore_barrier()

    def group(g, carry):
      p = lax.rem(g, 2)
      pn = 1 - p

      for b in range(NBUF):
        @pl.when(g > 0)
        def _():
          pltpu.make_async_copy(
              rows.at[b], acc_sh.at[idst.at[pn, b]], ssem[b]).wait()
        pltpu.async_copy(x_hbm.at[isrc.at[p, b]], rows.at[b], gsem[b])

      @pl.when(g + 1 < NGROUP)
      def _():
        nxt = (g + 1) * NBUF
        pltpu.async_copy(src_hbm.at[wid, pl.ds(nxt, NBUF)], isrc.at[pn],
                         xsem_s)
        pltpu.async_copy(dst_hbm.at[wid, pl.ds(nxt, NBUF)], idst.at[pn],
                         xsem_d)

      for b in range(NBUF):
        pltpu.make_async_copy(
            x_hbm.at[isrc.at[p, b]], rows.at[b], gsem[b]).wait()
        pltpu.async_copy(rows.at[b], acc_sh.at[idst.at[p, b]], ssem[b],
                         add=True)

      @pl.when(g + 1 < NGROUP)
      def _():
        nxt = (g + 1) * NBUF
        pltpu.make_async_copy(src_hbm.at[wid, pl.ds(nxt, NBUF)], isrc.at[pn],
                              xsem_s).wait()
        pltpu.make_async_copy(dst_hbm.at[wid, pl.ds(nxt, NBUF)], idst.at[pn],
                              xsem_d).wait()
      return carry

    lax.fori_loop(0, NGROUP, group, 0)
    pl_ = lax.rem(NGROUP - 1, 2)
    for b in range(NBUF):
      pltpu.make_async_copy(
          rows.at[b], acc_sh.at[idst.at[pl_, b]], ssem[b]).wait()
    plsc.subcore_barrier()

    pltpu.sync_copy(acc_sh.at[pl.ds(s * ZROWS, ZROWS)],
                    out_hbm.at[c, pl.ds(s * ZROWS, ZROWS)])

  return agg


_agg_cache = {}


def _agg16(*args):
  if 16 not in _agg_cache:
    _agg_cache[16] = _make_agg(16)
  return _agg_cache[16](*args)


def _agg64bf(*args):
  if 64 not in _agg_cache:
    _agg_cache[64] = _make_agg(64, jnp.bfloat16)
  return _agg_cache[64](*args)


def _d1_body(p1_ref, z_ref, wl_ref, wr_ref, b_ref, h1_ref, inv_ref):
  s = p1_ref[0] + p1_ref[1]
  cnt = s[:, 3:4]
  inv = 1.0 / jnp.maximum(cnt, 1.0)
  mean = s * inv
  h = (jnp.dot(mean, wl_ref[...], preferred_element_type=jnp.float32)
       + jnp.dot(z_ref[...], wr_ref[...], preferred_element_type=jnp.float32)
       + b_ref[...])
  h = jnp.maximum(h, 0.0)
  for p in range(2):
    h1_ref[p] = h[:, 64 * p:64 * (p + 1)].astype(jnp.bfloat16)
  inv_ref[...] = jnp.broadcast_to(inv, (ROW_TILE, 8))


def _d1(p1, zp, wl, wr, b):
  return pl.pallas_call(
      _d1_body,
      grid=(GRID,),
      in_specs=[
          pl.BlockSpec((NC, ROW_TILE, 16), lambda i: (0, i, 0)),
          pl.BlockSpec((ROW_TILE, 16), lambda i: (i, 0)),
          pl.BlockSpec((16, 128), lambda i: (0, 0)),
          pl.BlockSpec((16, 128), lambda i: (0, 0)),
          pl.BlockSpec((1, 128), lambda i: (0, 0)),
      ],
      out_specs=[
          pl.BlockSpec((2, ROW_TILE, 64), lambda i: (0, i, 0)),
          pl.BlockSpec((ROW_TILE, 8), lambda i: (i, 0)),
      ],
      out_shape=[
          jax.ShapeDtypeStruct((2, NP, 64), jnp.bfloat16),
          jax.ShapeDtypeStruct((NP, 8), jnp.float32),
      ],
  )(p1, zp, wl, wr, b)


def _d2_body(o0_ref, o1_ref, h1_ref, inv_ref,
             wl_ref, wr_ref, b_ref, w3l_ref, h2_ref, g3_ref):
  s2 = jnp.concatenate(
      [r[0].astype(jnp.float32) + r[1].astype(jnp.float32)
       for r in (o0_ref, o1_ref)], axis=1)
  mean2 = s2 * inv_ref[:, 0:1]
  h1 = jnp.concatenate(
      [h1_ref[p] for p in range(2)], axis=1).astype(jnp.float32)
  h2 = (jnp.dot(mean2, wl_ref[...], preferred_element_type=jnp.float32)
        + jnp.dot(h1, wr_ref[...], preferred_element_type=jnp.float32)
        + b_ref[...])
  h2 = jnp.maximum(h2, 0.0)
  h2_ref[...] = h2
  g = jnp.dot(h2, w3l_ref[...], preferred_element_type=jnp.float32)
  g3_ref[...] = g.astype(jnp.bfloat16)


def _d2(o2s, h1blk, inv8, wl, wr, b, w3l):
  spec_o = pl.BlockSpec((NC, ROW_TILE, 64), lambda i: (0, i, 0))
  return pl.pallas_call(
      _d2_body,
      grid=(GRID,),
      in_specs=[
          spec_o, spec_o,
          pl.BlockSpec((2, ROW_TILE, 64), lambda i: (0, i, 0)),
          pl.BlockSpec((ROW_TILE, 8), lambda i: (i, 0)),
          pl.BlockSpec((128, 128), lambda i: (0, 0)),
          pl.BlockSpec((128, 128), lambda i: (0, 0)),
          pl.BlockSpec((1, 128), lambda i: (0, 0)),
          pl.BlockSpec((128, 64), lambda i: (0, 0)),
      ],
      out_specs=[
          pl.BlockSpec((ROW_TILE, 128), lambda i: (i, 0)),
          pl.BlockSpec((ROW_TILE, 64), lambda i: (i, 0)),
      ],
      out_shape=[
          jax.ShapeDtypeStruct((NP, 128), jnp.float32),
          jax.ShapeDtypeStruct((NP, 64), jnp.bfloat16),
      ],
  )(*o2s, h1blk, inv8, wl, wr, b, w3l)


def _d3_body(o0_ref, h2_ref, inv_ref, w3r_ref, b3_ref,
             w4_ref, b4_ref, out_ref):
  sg = o0_ref[0].astype(jnp.float32) + o0_ref[1].astype(jnp.float32)
  h3 = (sg * inv_ref[:, 0:1]
        + jnp.dot(h2_ref[...], w3r_ref[...], preferred_element_type=jnp.float32)
        + b3_ref[...])
  h3 = jnp.maximum(h3, 0.0)
  out_ref[...] = (jnp.dot(h3, w4_ref[...], preferred_element_type=jnp.float32)
                  + b4_ref[...])


def _d3(o3, h2, inv8, w3r, b3, w4, b4):
  spec_o = pl.BlockSpec((NC, ROW_TILE, 64), lambda i: (0, i, 0))
  return pl.pallas_call(
      _d3_body,
      grid=(GRID,),
      in_specs=[
          spec_o,
          pl.BlockSpec((ROW_TILE, 128), lambda i: (i, 0)),
          pl.BlockSpec((ROW_TILE, 8), lambda i: (i, 0)),
          pl.BlockSpec((128, 64), lambda i: (0, 0)),
          pl.BlockSpec((1, 64), lambda i: (0, 0)),
          pl.BlockSpec((64, 40), lambda i: (0, 0)),
          pl.BlockSpec((1, 40), lambda i: (0, 0)),
      ],
      out_specs=pl.BlockSpec((ROW_TILE, 40), lambda i: (i, 0)),
      out_shape=jax.ShapeDtypeStruct((NP, 40), jnp.float32),
  )(o3, h2, inv8, w3r, b3, w4, b4)


@jax.jit
def kernel(z, edge_index, W1l, b1, W1r, W2l, b2, W2r, W3l, b3, W3r, W4, b4):
  ei = edge_index.astype(jnp.int32)
  pad = EP - N_EDGES
  src = jnp.concatenate([ei[0], jnp.zeros((pad,), jnp.int32)])
  dst = jnp.concatenate([ei[1], jnp.full((pad,), TRASH, jnp.int32)])
  src = src.reshape(NW, NCHUNK, CHUNK)
  dst = dst.reshape(NW, NCHUNK, CHUNK)

  zp = jnp.zeros((NP, 16), jnp.float32)
  zp = zp.at[:N_NODES, :3].set(z)
  zp = zp.at[:N_NODES, 3].set(1.0)

  w1l = jnp.zeros((16, 128), jnp.float32).at[:3].set(W1l.T)
  w1r = jnp.zeros((16, 128), jnp.float32).at[:3].set(W1r.T)

  zer16 = jnp.zeros((ZROWS, 16), jnp.float32)
  zer64 = jnp.zeros((ZROWS, 64), jnp.bfloat16)

  p1 = _agg16(zp, src, dst, zer16)
  h1blk, inv8 = _d1(p1, zp, w1l, w1r, b1.reshape(1, 128))

  o2s = [_agg64bf(h1blk[p], src, dst, zer64) for p in range(2)]
  h2, g3blk = _d2(o2s, h1blk, inv8, W2l.T, W2r.T, b2.reshape(1, 128), W3l.T)

  o3 = _agg64bf(g3blk, src, dst, zer64)
  out = _d3(o3, h2, inv8, W3r.T, b3.reshape(1, 64), W4.T, b4.reshape(1, 40))
  return out[:N_NODES]
